# baseline (speedup 1.0000x reference)
import functools

import jax
import jax.numpy as jnp
from jax import lax
from jax.experimental import pallas as pl
from jax.experimental.pallas import tpu as pltpu
from jax.experimental.pallas import tpu_sc as plsc

F32 = jnp.float32

N = 10000
E = 320000
DF = 128
DE = 16
H = 100
HP = 128
NC = 2
NS = 16
NW = NC * NS
EP = 327680
EPW = EP // NW
CH = 128
NCHUNK = EPW // CH
CH_MS = 64
NCH_MS = EPW // CH_MS
NP = 10240
RPS = NP // NS
BE = 2048
GE = EP // BE

_mesh = plsc.VectorSubcoreMesh(core_axis_name="c", subcore_axis_name="s")


def _zero_vmem_rows(buf, rows):
    def body(i, _):
        for j in range(HP // 16):
            buf[i, pl.ds(j * 16, 16)] = jnp.zeros((16,), F32)
        return 0
    lax.fori_loop(0, rows, body, 0)


def _relu_add_rows(dst_buf, a_buf, rows):
    def body(i, _):
        for j in range(HP // 16):
            sl = pl.ds(j * 16, 16)
            dst_buf[i, sl] = jnp.maximum(a_buf[i, sl] + dst_buf[i, sl], 0.0)
        return 0
    lax.fori_loop(0, rows, body, 0)


def _sc_msg_scatter(x0, ea0, src_p, dst_p, gather_x):
    @functools.partial(
        pl.kernel,
        out_type=jax.ShapeDtypeStruct((NC * NP, HP), F32),
        mesh=_mesh,
        scratch_types=[
            pltpu.VMEM((1, CH_MS), jnp.int32),
            pltpu.VMEM((1, CH_MS), jnp.int32),
            pltpu.VMEM((1, CH_MS), jnp.int32),
            pltpu.VMEM((1, CH_MS), jnp.int32),
            pltpu.VMEM((CH_MS, HP), F32),
            pltpu.VMEM((CH_MS, HP), F32),
            pltpu.VMEM((CH_MS, HP), F32),
            pltpu.VMEM((CH_MS, HP), F32),
            pltpu.SemaphoreType.DMA,
            pltpu.SemaphoreType.DMA,
            pltpu.SemaphoreType.DMA,
            pltpu.SemaphoreType.DMA,
            pltpu.SemaphoreType.DMA,
            pltpu.SemaphoreType.DMA,
            pltpu.SemaphoreType.DMA,
            pltpu.SemaphoreType.DMA,
            pltpu.VMEM_SHARED((NP, HP), F32),
        ],
        interpret=False,
    )
    def k(x_hbm, ea_hbm, src_hbm, dst_hbm, out_hbm,
          is0, is1, id0, id1, xr0, xr1, eb0, eb1,
          sg0, sg1, se0, se1, ss0, ss1, sd0, sd1, agg_sh):
        c = lax.axis_index("c")
        s = lax.axis_index("s")
        gwid = c * NS + s
        isb = (is0, is1)
        idb = (id0, id1)
        xr = (xr0, xr1)
        eb = (eb0, eb1)
        sg = (sg0, sg1)
        se = (se0, se1)
        ss = (ss0, ss1)
        sd = (sd0, sd1)

        _zero_vmem_rows(xr0, CH_MS)
        def zc(t, _):
            pltpu.sync_copy(xr0, agg_sh.at[pl.ds(s * RPS + t * CH_MS,
                                                 CH_MS)])
            return 0
        lax.fori_loop(0, RPS // CH_MS, zc, 0)
        plsc.subcore_barrier()

        def start_idx(kk, b):
            base = gwid * EPW + kk * CH_MS
            pltpu.async_copy(dst_hbm.at[pl.ds(base, CH_MS)], idb[b].at[0],
                             sd[b])
            if gather_x:
                pltpu.async_copy(src_hbm.at[pl.ds(base, CH_MS)],
                                 isb[b].at[0], ss[b])

        def wait_idx(kk, b):
            base = gwid * EPW + kk * CH_MS
            pltpu.make_async_copy(dst_hbm.at[pl.ds(base, CH_MS)],
                                  idb[b].at[0], sd[b]).wait()
            if gather_x:
                pltpu.make_async_copy(src_hbm.at[pl.ds(base, CH_MS)],
                                      isb[b].at[0], ss[b]).wait()

        def start_in(kk, b):
            base = gwid * EPW + kk * CH_MS
            if gather_x:
                pltpu.async_copy(x_hbm.at[isb[b].at[0]], xr[b], sg[b])
            else:
                pltpu.async_copy(x_hbm.at[pl.ds(base, CH_MS)], xr[b], sg[b])
            pltpu.async_copy(ea_hbm.at[pl.ds(base, CH_MS)], eb[b], se[b])

        def wait_in(kk, b):
            base = gwid * EPW + kk * CH_MS
            if gather_x:
                pltpu.make_async_copy(x_hbm.at[isb[b].at[0]], xr[b],
                                      sg[b]).wait()
            else:
                pltpu.make_async_copy(x_hbm.at[pl.ds(base, CH_MS)], xr[b],
                                      sg[b]).wait()
            pltpu.make_async_copy(ea_hbm.at[pl.ds(base, CH_MS)], eb[b],
                                  se[b]).wait()

        start_idx(0, 0)
        wait_idx(0, 0)
        start_in(0, 0)
        start_idx(1, 1)

        def body(t, _):
            for b in range(2):
                kk = t * 2 + b
                nb = 1 - b
                @pl.when(kk + 1 < NCH_MS)
                def _():
                    wait_idx(kk + 1, nb)
                    start_in(kk + 1, nb)
                wait_in(kk, b)
                _relu_add_rows(eb[b], xr[b], CH_MS)
                pltpu.sync_copy(eb[b], agg_sh.at[idb[b].at[0]], add=True)
                @pl.when(kk + 2 < NCH_MS)
                def _():
                    start_idx(kk + 2, b)
            return 0
        lax.fori_loop(0, NCH_MS // 2, body, 0)
        plsc.subcore_barrier()
        pltpu.sync_copy(agg_sh.at[pl.ds(s * RPS, RPS)],
                        out_hbm.at[pl.ds(c * NP + s * RPS, RPS)])

    return k(x0, ea0, src_p, dst_p)


def _sc_gather2(x0, src_p, dst_p):
    @functools.partial(
        pl.kernel,
        out_type=(jax.ShapeDtypeStruct((EP, HP), F32),
                  jax.ShapeDtypeStruct((EP, HP), F32)),
        mesh=_mesh,
        scratch_types=[
            pltpu.VMEM((1, CH_MS), jnp.int32),
            pltpu.VMEM((1, CH_MS), jnp.int32),
            pltpu.VMEM((1, CH_MS), jnp.int32),
            pltpu.VMEM((1, CH_MS), jnp.int32),
            pltpu.VMEM((CH_MS, HP), F32),
            pltpu.VMEM((CH_MS, HP), F32),
            pltpu.VMEM((CH_MS, HP), F32),
            pltpu.VMEM((CH_MS, HP), F32),
            pltpu.SemaphoreType.DMA,
            pltpu.SemaphoreType.DMA,
            pltpu.SemaphoreType.DMA,
            pltpu.SemaphoreType.DMA,
            pltpu.SemaphoreType.DMA,
            pltpu.SemaphoreType.DMA,
            pltpu.SemaphoreType.DMA,
            pltpu.SemaphoreType.DMA,
            pltpu.SemaphoreType.DMA,
            pltpu.SemaphoreType.DMA,
            pltpu.SemaphoreType.DMA,
            pltpu.SemaphoreType.DMA,
            pltpu.VMEM_SHARED((N, HP), F32),
        ],
        interpret=False,
    )
    def k(x_hbm, src_hbm, dst_hbm, xs_hbm, xd_hbm,
          is0, is1, id0, id1, bs0, bs1, bd0, bd1,
          gs0, gs1, gd0, gd1, ws0, ws1, wd0, wd1,
          ss0, ss1, sd0, sd1, x_sh):
        c = lax.axis_index("c")
        s = lax.axis_index("s")
        gwid = c * NS + s
        isb = (is0, is1)
        idb = (id0, id1)
        bs = (bs0, bs1)
        bd = (bd0, bd1)
        gs = (gs0, gs1)
        gd = (gd0, gd1)
        ws = (ws0, ws1)
        wd = (wd0, wd1)
        ss = (ss0, ss1)
        sd = (sd0, sd1)

        pltpu.sync_copy(x_hbm.at[pl.ds(s * 624, 640)],
                        x_sh.at[pl.ds(s * 624, 640)])
        plsc.subcore_barrier()

        def start_idx(kk, b):
            base = gwid * EPW + kk * CH_MS
            pltpu.async_copy(src_hbm.at[pl.ds(base, CH_MS)], isb[b].at[0],
                             ss[b])
            pltpu.async_copy(dst_hbm.at[pl.ds(base, CH_MS)], idb[b].at[0],
                             sd[b])

        def wait_idx(kk, b):
            base = gwid * EPW + kk * CH_MS
            pltpu.make_async_copy(src_hbm.at[pl.ds(base, CH_MS)],
                                  isb[b].at[0], ss[b]).wait()
            pltpu.make_async_copy(dst_hbm.at[pl.ds(base, CH_MS)],
                                  idb[b].at[0], sd[b]).wait()

        def start_in(kk, b):
            pltpu.async_copy(x_sh.at[isb[b].at[0]], bs[b], gs[b])
            pltpu.async_copy(x_sh.at[idb[b].at[0]], bd[b], gd[b])

        def wait_in(kk, b):
            pltpu.make_async_copy(x_sh.at[isb[b].at[0]], bs[b], gs[b]).wait()
            pltpu.make_async_copy(x_sh.at[idb[b].at[0]], bd[b], gd[b]).wait()

        def wait_out(kk, b):
            base = gwid * EPW + kk * CH_MS
            pltpu.make_async_copy(bs[b], xs_hbm.at[pl.ds(base, CH_MS)],
                                  ws[b]).wait()
            pltpu.make_async_copy(bd[b], xd_hbm.at[pl.ds(base, CH_MS)],
                                  wd[b]).wait()

        start_idx(0, 0)
        wait_idx(0, 0)
        start_in(0, 0)
        start_idx(1, 1)

        def body(t, _):
            for b in range(2):
                kk = t * 2 + b
                nb = 1 - b
                @pl.when(kk + 1 < NCH_MS)
                def _():
                    wait_idx(kk + 1, nb)
                    @pl.when(kk >= 1)
                    def _():
                        wait_out(kk - 1, nb)
                    start_in(kk + 1, nb)
                wait_in(kk, b)
                @pl.when(kk + 2 < NCH_MS)
                def _():
                    start_idx(kk + 2, b)
                base = gwid * EPW + kk * CH_MS
                pltpu.async_copy(bs[b], xs_hbm.at[pl.ds(base, CH_MS)], ws[b])
                pltpu.async_copy(bd[b], xd_hbm.at[pl.ds(base, CH_MS)], wd[b])
            return 0
        lax.fori_loop(0, NCH_MS // 2, body, 0)
        wait_out(NCH_MS - 2, 0)
        wait_out(NCH_MS - 1, 1)

    return k(x0, src_p, dst_p)


def _proj_node(x, wT, b):
    def k(x_ref, w_ref, b_ref, o_ref):
        o_ref[...] = jnp.dot(x_ref[...], w_ref[...],
                             preferred_element_type=F32) + b_ref[...]
    return pl.pallas_call(
        k,
        out_shape=jax.ShapeDtypeStruct((N, HP), F32),
        interpret=False,
    )(x, wT, b)


def _proj_edge(ea_in, wT, b):
    def k(a_ref, w_ref, b_ref, o_ref):
        o_ref[...] = jnp.dot(a_ref[...], w_ref[...],
                             preferred_element_type=F32) + b_ref[...]
    return pl.pallas_call(
        k,
        grid=(GE,),
        in_specs=[
            pl.BlockSpec((BE, DE), lambda i: (i, 0)),
            pl.BlockSpec((DE, HP), lambda i: (0, 0)),
            pl.BlockSpec((1, HP), lambda i: (0, 0)),
        ],
        out_specs=pl.BlockSpec((BE, HP), lambda i: (i, 0)),
        out_shape=jax.ShapeDtypeStruct((EP, HP), F32),
        interpret=False,
    )(ea_in, wT, b)


def _node_mlp(x, agg, w1T, b1, w2T, b2, g, bt):
    def k(x_ref, a_ref, w1_ref, b1_ref, w2_ref, b2_ref, g_ref, bt_ref, o_ref):
        xv = x_ref[...]
        a = a_ref[0:N, :] + a_ref[NP:NP + N, :]
        h = xv + a
        h = jnp.maximum(jnp.dot(h, w1_ref[...], preferred_element_type=F32)
                        + b1_ref[...], 0.0)
        h = jnp.dot(h, w2_ref[...], preferred_element_type=F32) + b2_ref[...]
        m = jnp.mean(h, axis=0, keepdims=True)
        v = jnp.mean((h - m) ** 2, axis=0, keepdims=True)
        hn = (h - m) * lax.rsqrt(v + 1e-5) * g_ref[...] + bt_ref[...]
        o_ref[...] = (xv + jnp.maximum(hn, 0.0)) * 0.5
    return pl.pallas_call(
        k,
        out_shape=jax.ShapeDtypeStruct((N, HP), F32),
        interpret=False,
    )(x, agg, w1T, b1, w2T, b2, g, bt)


def _edge_mlp(xs, xd, ea, w1a, w1b, w1c, b1, w2T, b2):
    def k(xs_ref, xd_ref, ea_ref, wa_ref, wb_ref, wc_ref, b1_ref,
          w2_ref, b2_ref, o_ref):
        eav = ea_ref[...]
        t = (jnp.dot(xs_ref[...], wa_ref[...], preferred_element_type=F32)
             + jnp.dot(xd_ref[...], wb_ref[...], preferred_element_type=F32)
             + jnp.dot(eav, wc_ref[...], preferred_element_type=F32)
             + b1_ref[...])
        t = jnp.maximum(t, 0.0)
        o_ref[...] = eav + (jnp.dot(t, w2_ref[...], preferred_element_type=F32)
                            + b2_ref[...]) * 0.5
    wspec = pl.BlockSpec((HP, HP), lambda i: (0, 0))
    bspec = pl.BlockSpec((1, HP), lambda i: (0, 0))
    espec = pl.BlockSpec((BE, HP), lambda i: (i, 0))
    return pl.pallas_call(
        k,
        grid=(GE,),
        in_specs=[espec, espec, espec, wspec, wspec, wspec, bspec,
                  wspec, bspec],
        out_specs=espec,
        out_shape=jax.ShapeDtypeStruct((EP, HP), F32),
        interpret=False,
    )(xs, xd, ea, w1a, w1b, w1c, b1, w2T, b2)


def _final_mlp(xs, xd, ea, w1a, w1b, w1c, b1, w2T, b2,
               m1a, m1b, m1c, bm1, m2T, bm2, m3, bm3):
    def k(xs_ref, xd_ref, ea_ref, wa_ref, wb_ref, wc_ref, b1_ref,
          w2_ref, b2_ref, ma_ref, mb_ref, mc_ref, bm1_ref,
          m2_ref, bm2_ref, m3_ref, bm3_ref, o_ref):
        xsv = xs_ref[...]
        xdv = xd_ref[...]
        eav = ea_ref[...]
        t = (jnp.dot(xsv, wa_ref[...], preferred_element_type=F32)
             + jnp.dot(xdv, wb_ref[...], preferred_element_type=F32)
             + jnp.dot(eav, wc_ref[...], preferred_element_type=F32)
             + b1_ref[...])
        t = jnp.maximum(t, 0.0)
        ea2 = eav + (jnp.dot(t, w2_ref[...], preferred_element_type=F32)
                     + b2_ref[...]) * 0.5
        o1 = (jnp.dot(xsv, ma_ref[...], preferred_element_type=F32)
              + jnp.dot(xdv, mb_ref[...], preferred_element_type=F32)
              + jnp.dot(ea2, mc_ref[...], preferred_element_type=F32)
              + bm1_ref[...])
        o1 = jnp.maximum(o1, 0.0)
        o2 = jnp.maximum(jnp.dot(o1, m2_ref[...], preferred_element_type=F32)
                         + bm2_ref[...], 0.0)
        o_ref[...] = lax.dot_general(m3_ref[...], o2, (((1,), (1,)), ((), ())),
                                     preferred_element_type=F32) + bm3_ref[...]
    wspec = pl.BlockSpec((HP, HP), lambda i: (0, 0))
    bspec = pl.BlockSpec((1, HP), lambda i: (0, 0))
    espec = pl.BlockSpec((BE, HP), lambda i: (i, 0))
    return pl.pallas_call(
        k,
        grid=(GE,),
        in_specs=[espec, espec, espec,
                  wspec, wspec, wspec, bspec, wspec, bspec,
                  wspec, wspec, wspec, bspec, wspec, bspec,
                  pl.BlockSpec((8, HP), lambda i: (0, 0)),
                  pl.BlockSpec((8, 1), lambda i: (0, 0))],
        out_specs=pl.BlockSpec((8, BE), lambda i: (0, i)),
        out_shape=jax.ShapeDtypeStruct((8, EP), F32),
        interpret=False,
    )(xs, xd, ea, w1a, w1b, w1c, b1, w2T, b2,
      m1a, m1b, m1c, bm1, m2T, bm2, m3, bm3)


def _pad2(w, r, c):
    return jnp.zeros((r, c), F32).at[:w.shape[0], :w.shape[1]].set(w)


def _pad_cat3(w, r):
    z = jnp.zeros((r, 3 * HP), F32)
    for t in range(3):
        z = z.at[:w.shape[0], HP * t:HP * t + H].set(w[:, H * t:H * t + H])
    return z


def _padb(b, n=HP):
    return jnp.zeros((1, n), F32).at[0, :b.shape[0]].set(b)


def kernel(x, edge_index, edge_attr, W_node, b_node, W_edge, b_edge,
           W1, b1, W2, b2, We1, be1, We2, be2, gamma, beta,
           Wm1, bm1, Wm2, bm2, Wm3, bm3):
    pad = EP - E
    padi = jnp.arange(pad, dtype=jnp.int32)
    src_flat = jnp.concatenate([edge_index[0], padi % N])
    dst_flat = jnp.concatenate([edge_index[1], N + (padi % (NP - N))])
    dst_gf = jnp.concatenate([edge_index[1], padi % N])
    ea_in = jnp.zeros((EP, DE), F32).at[:E].set(edge_attr)

    wnT = _pad2(W_node, HP, DF).T
    weT = _pad2(W_edge, HP, DE).T
    bnp = _padb(b_node)
    bep = _padb(b_edge)

    x0 = _proj_node(x, wnT, bnp)
    ea0 = _proj_edge(ea_in, weT, bep)

    xc = x0
    eac = ea0
    xs = xd = None
    for i in range(2):
        w1T = _pad2(W1[i], HP, HP).T
        w2T = _pad2(W2[i], HP, HP).T
        b1p = _padb(b1[i])
        b2p = _padb(b2[i])
        gp = _padb(gamma[i])
        btp = _padb(beta[i])
        if i == 0:
            agg = _sc_msg_scatter(xc, eac, src_flat, dst_flat, gather_x=True)
        else:
            agg = _sc_msg_scatter(xs, eac, src_flat, dst_flat, gather_x=False)
        xc = _node_mlp(xc, agg, w1T, b1p, w2T, b2p, gp, btp)
        xs, xd = _sc_gather2(xc, src_flat, dst_gf)
        if i == 0:
            w1a = _pad2(We1[i][:, 0:H], HP, HP).T
            w1b = _pad2(We1[i][:, H:2 * H], HP, HP).T
            w1c = _pad2(We1[i][:, 2 * H:3 * H], HP, HP).T
            we2T = _pad2(We2[i], HP, HP).T
            eac = _edge_mlp(xs, xd, eac, w1a, w1b, w1c, _padb(be1[i]),
                            we2T, _padb(be2[i]))

    m1a = _pad2(Wm1[:, 0:H], HP, HP).T
    m1b = _pad2(Wm1[:, H:2 * H], HP, HP).T
    m1c = _pad2(Wm1[:, 2 * H:3 * H], HP, HP).T
    m2T = _pad2(Wm2, HP, HP).T
    m3 = _pad2(Wm3, 8, HP)
    bm3p = jnp.zeros((8, 1), F32).at[:2, 0].set(bm3)

    w1a = _pad2(We1[1][:, 0:H], HP, HP).T
    w1b = _pad2(We1[1][:, H:2 * H], HP, HP).T
    w1c = _pad2(We1[1][:, 2 * H:3 * H], HP, HP).T
    we2T = _pad2(We2[1], HP, HP).T

    outT = _final_mlp(xs, xd, eac, w1a, w1b, w1c, _padb(be1[1]),
                      we2T, _padb(be2[1]),
                      m1a, m1b, m1c, _padb(bm1), m2T, _padb(bm2), m3, bm3p)
    return outT[:2, :E].T


# speedup vs baseline: 1.1233x; 1.1233x over previous
import functools

import jax
import jax.numpy as jnp
from jax import lax
from jax.experimental import pallas as pl
from jax.experimental.pallas import tpu as pltpu
from jax.experimental.pallas import tpu_sc as plsc

F32 = jnp.float32

N = 10000
E = 320000
DF = 128
DE = 16
H = 100
HP = 128
NC = 2
NS = 16
NW = NC * NS
EP = 327680
EPW = EP // NW
CH = 128
NCHUNK = EPW // CH
CH_MS = 64
NCH_MS = EPW // CH_MS
NP = 10240
RPS = NP // NS
BE = 2048
GE = EP // BE

_mesh = plsc.VectorSubcoreMesh(core_axis_name="c", subcore_axis_name="s")


def _zero_vmem_rows(buf, rows):
    def body(i, _):
        for j in range(HP // 16):
            buf[i, pl.ds(j * 16, 16)] = jnp.zeros((16,), F32)
        return 0
    lax.fori_loop(0, rows, body, 0)


def _relu_add_rows(dst_buf, a_buf, rows):
    def body(i, _):
        for j in range(HP // 16):
            sl = pl.ds(j * 16, 16)
            dst_buf[i, sl] = jnp.maximum(a_buf[i, sl] + dst_buf[i, sl], 0.0)
        return 0
    lax.fori_loop(0, rows, body, 0)


def _sc_msg_scatter(x0, ea0, src_p, dst_p, gather_x):
    @functools.partial(
        pl.kernel,
        out_type=jax.ShapeDtypeStruct((NC * NP, HP), F32),
        mesh=_mesh,
        scratch_types=[
            pltpu.VMEM((1, CH_MS), jnp.int32),
            pltpu.VMEM((1, CH_MS), jnp.int32),
            pltpu.VMEM((1, CH_MS), jnp.int32),
            pltpu.VMEM((1, CH_MS), jnp.int32),
            pltpu.VMEM((CH_MS, HP), F32),
            pltpu.VMEM((CH_MS, HP), F32),
            pltpu.VMEM((CH_MS, HP), F32),
            pltpu.VMEM((CH_MS, HP), F32),
            pltpu.SemaphoreType.DMA,
            pltpu.SemaphoreType.DMA,
            pltpu.SemaphoreType.DMA,
            pltpu.SemaphoreType.DMA,
            pltpu.SemaphoreType.DMA,
            pltpu.SemaphoreType.DMA,
            pltpu.SemaphoreType.DMA,
            pltpu.SemaphoreType.DMA,
            pltpu.VMEM_SHARED((NP, HP), F32),
        ],
        interpret=False,
    )
    def k(x_hbm, ea_hbm, src_hbm, dst_hbm, out_hbm,
          is0, is1, id0, id1, xr0, xr1, eb0, eb1,
          sg0, sg1, se0, se1, ss0, ss1, sd0, sd1, agg_sh):
        c = lax.axis_index("c")
        s = lax.axis_index("s")
        gwid = c * NS + s
        isb = (is0, is1)
        idb = (id0, id1)
        xr = (xr0, xr1)
        eb = (eb0, eb1)
        sg = (sg0, sg1)
        se = (se0, se1)
        ss = (ss0, ss1)
        sd = (sd0, sd1)

        _zero_vmem_rows(xr0, CH_MS)
        def zc(t, _):
            pltpu.sync_copy(xr0, agg_sh.at[pl.ds(s * RPS + t * CH_MS,
                                                 CH_MS)])
            return 0
        lax.fori_loop(0, RPS // CH_MS, zc, 0)
        plsc.subcore_barrier()

        def start_idx(kk, b):
            base = gwid * EPW + kk * CH_MS
            pltpu.async_copy(dst_hbm.at[pl.ds(base, CH_MS)], idb[b].at[0],
                             sd[b])
            if gather_x:
                pltpu.async_copy(src_hbm.at[pl.ds(base, CH_MS)],
                                 isb[b].at[0], ss[b])

        def wait_idx(kk, b):
            base = gwid * EPW + kk * CH_MS
            pltpu.make_async_copy(dst_hbm.at[pl.ds(base, CH_MS)],
                                  idb[b].at[0], sd[b]).wait()
            if gather_x:
                pltpu.make_async_copy(src_hbm.at[pl.ds(base, CH_MS)],
                                      isb[b].at[0], ss[b]).wait()

        def start_in(kk, b):
            base = gwid * EPW + kk * CH_MS
            if gather_x:
                pltpu.async_copy(x_hbm.at[isb[b].at[0]], xr[b], sg[b])
            else:
                pltpu.async_copy(x_hbm.at[pl.ds(base, CH_MS)], xr[b], sg[b])
            pltpu.async_copy(ea_hbm.at[pl.ds(base, CH_MS)], eb[b], se[b])

        def wait_in(kk, b):
            base = gwid * EPW + kk * CH_MS
            if gather_x:
                pltpu.make_async_copy(x_hbm.at[isb[b].at[0]], xr[b],
                                      sg[b]).wait()
            else:
                pltpu.make_async_copy(x_hbm.at[pl.ds(base, CH_MS)], xr[b],
                                      sg[b]).wait()
            pltpu.make_async_copy(ea_hbm.at[pl.ds(base, CH_MS)], eb[b],
                                  se[b]).wait()

        start_idx(0, 0)
        wait_idx(0, 0)
        start_in(0, 0)
        start_idx(1, 1)

        def body(t, _):
            for b in range(2):
                kk = t * 2 + b
                nb = 1 - b
                @pl.when(kk + 1 < NCH_MS)
                def _():
                    wait_idx(kk + 1, nb)
                    start_in(kk + 1, nb)
                wait_in(kk, b)
                _relu_add_rows(eb[b], xr[b], CH_MS)
                pltpu.sync_copy(eb[b], agg_sh.at[idb[b].at[0]], add=True)
                @pl.when(kk + 2 < NCH_MS)
                def _():
                    start_idx(kk + 2, b)
            return 0
        lax.fori_loop(0, NCH_MS // 2, body, 0)
        plsc.subcore_barrier()
        pltpu.sync_copy(agg_sh.at[pl.ds(s * RPS, RPS)],
                        out_hbm.at[pl.ds(c * NP + s * RPS, RPS)])

    return k(x0, ea0, src_p, dst_p)


def _sc_gather2(x0, src_p, dst_p):
    @functools.partial(
        pl.kernel,
        out_type=(jax.ShapeDtypeStruct((EP, HP), F32),
                  jax.ShapeDtypeStruct((EP, HP), F32)),
        mesh=_mesh,
        scratch_types=[
            pltpu.VMEM((1, CH_MS), jnp.int32),
            pltpu.VMEM((1, CH_MS), jnp.int32),
            pltpu.VMEM((1, CH_MS), jnp.int32),
            pltpu.VMEM((1, CH_MS), jnp.int32),
            pltpu.VMEM((CH_MS, HP), F32),
            pltpu.VMEM((CH_MS, HP), F32),
            pltpu.VMEM((CH_MS, HP), F32),
            pltpu.VMEM((CH_MS, HP), F32),
            pltpu.SemaphoreType.DMA,
            pltpu.SemaphoreType.DMA,
            pltpu.SemaphoreType.DMA,
            pltpu.SemaphoreType.DMA,
            pltpu.SemaphoreType.DMA,
            pltpu.SemaphoreType.DMA,
            pltpu.SemaphoreType.DMA,
            pltpu.SemaphoreType.DMA,
            pltpu.SemaphoreType.DMA,
            pltpu.SemaphoreType.DMA,
            pltpu.SemaphoreType.DMA,
            pltpu.SemaphoreType.DMA,
            pltpu.VMEM_SHARED((N, HP), F32),
        ],
        interpret=False,
    )
    def k(x_hbm, src_hbm, dst_hbm, xs_hbm, xd_hbm,
          is0, is1, id0, id1, bs0, bs1, bd0, bd1,
          gs0, gs1, gd0, gd1, ws0, ws1, wd0, wd1,
          ss0, ss1, sd0, sd1, x_sh):
        c = lax.axis_index("c")
        s = lax.axis_index("s")
        gwid = c * NS + s
        isb = (is0, is1)
        idb = (id0, id1)
        bs = (bs0, bs1)
        bd = (bd0, bd1)
        gs = (gs0, gs1)
        gd = (gd0, gd1)
        ws = (ws0, ws1)
        wd = (wd0, wd1)
        ss = (ss0, ss1)
        sd = (sd0, sd1)

        pltpu.sync_copy(x_hbm.at[pl.ds(s * 624, 640)],
                        x_sh.at[pl.ds(s * 624, 640)])
        plsc.subcore_barrier()

        def start_idx(kk, b):
            base = gwid * EPW + kk * CH_MS
            pltpu.async_copy(src_hbm.at[pl.ds(base, CH_MS)], isb[b].at[0],
                             ss[b])
            pltpu.async_copy(dst_hbm.at[pl.ds(base, CH_MS)], idb[b].at[0],
                             sd[b])

        def wait_idx(kk, b):
            base = gwid * EPW + kk * CH_MS
            pltpu.make_async_copy(src_hbm.at[pl.ds(base, CH_MS)],
                                  isb[b].at[0], ss[b]).wait()
            pltpu.make_async_copy(dst_hbm.at[pl.ds(base, CH_MS)],
                                  idb[b].at[0], sd[b]).wait()

        def start_in(kk, b):
            pltpu.async_copy(x_sh.at[isb[b].at[0]], bs[b], gs[b])
            pltpu.async_copy(x_sh.at[idb[b].at[0]], bd[b], gd[b])

        def wait_in(kk, b):
            pltpu.make_async_copy(x_sh.at[isb[b].at[0]], bs[b], gs[b]).wait()
            pltpu.make_async_copy(x_sh.at[idb[b].at[0]], bd[b], gd[b]).wait()

        def wait_out(kk, b):
            base = gwid * EPW + kk * CH_MS
            pltpu.make_async_copy(bs[b], xs_hbm.at[pl.ds(base, CH_MS)],
                                  ws[b]).wait()
            pltpu.make_async_copy(bd[b], xd_hbm.at[pl.ds(base, CH_MS)],
                                  wd[b]).wait()

        start_idx(0, 0)
        wait_idx(0, 0)
        start_in(0, 0)
        start_idx(1, 1)

        def body(t, _):
            for b in range(2):
                kk = t * 2 + b
                nb = 1 - b
                @pl.when(kk + 1 < NCH_MS)
                def _():
                    wait_idx(kk + 1, nb)
                    @pl.when(kk >= 1)
                    def _():
                        wait_out(kk - 1, nb)
                    start_in(kk + 1, nb)
                wait_in(kk, b)
                @pl.when(kk + 2 < NCH_MS)
                def _():
                    start_idx(kk + 2, b)
                base = gwid * EPW + kk * CH_MS
                pltpu.async_copy(bs[b], xs_hbm.at[pl.ds(base, CH_MS)], ws[b])
                pltpu.async_copy(bd[b], xd_hbm.at[pl.ds(base, CH_MS)], wd[b])
            return 0
        lax.fori_loop(0, NCH_MS // 2, body, 0)
        wait_out(NCH_MS - 2, 0)
        wait_out(NCH_MS - 1, 1)

    return k(x0, src_p, dst_p)


def _proj_node(x, wT, b):
    def k(x_ref, w_ref, b_ref, o_ref):
        o_ref[...] = jnp.dot(x_ref[...], w_ref[...],
                             preferred_element_type=F32) + b_ref[...]
    return pl.pallas_call(
        k,
        out_shape=jax.ShapeDtypeStruct((N, HP), F32),
        interpret=False,
    )(x, wT, b)


def _proj_edge(ea_t, wT, b):
    def k(a_ref, w_ref, b_ref, o_ref):
        o_ref[...] = lax.dot_general(
            a_ref[...], w_ref[...], (((0,), (0,)), ((), ())),
            preferred_element_type=F32) + b_ref[...]
    return pl.pallas_call(
        k,
        grid=(GE,),
        in_specs=[
            pl.BlockSpec((DE, BE), lambda i: (0, i)),
            pl.BlockSpec((DE, HP), lambda i: (0, 0)),
            pl.BlockSpec((1, HP), lambda i: (0, 0)),
        ],
        out_specs=pl.BlockSpec((BE, HP), lambda i: (i, 0)),
        out_shape=jax.ShapeDtypeStruct((EP, HP), F32),
        interpret=False,
    )(ea_t, wT, b)


def _node_mlp(x, agg, w1T, b1, w2T, b2, g, bt):
    def k(x_ref, a_ref, w1_ref, b1_ref, w2_ref, b2_ref, g_ref, bt_ref, o_ref):
        xv = x_ref[...]
        a = a_ref[0:N, :] + a_ref[NP:NP + N, :]
        h = xv + a
        h = jnp.maximum(jnp.dot(h, w1_ref[...], preferred_element_type=F32)
                        + b1_ref[...], 0.0)
        h = jnp.dot(h, w2_ref[...], preferred_element_type=F32) + b2_ref[...]
        m = jnp.mean(h, axis=0, keepdims=True)
        v = jnp.mean((h - m) ** 2, axis=0, keepdims=True)
        hn = (h - m) * lax.rsqrt(v + 1e-5) * g_ref[...] + bt_ref[...]
        o_ref[...] = (xv + jnp.maximum(hn, 0.0)) * 0.5
    return pl.pallas_call(
        k,
        out_shape=jax.ShapeDtypeStruct((N, HP), F32),
        interpret=False,
    )(x, agg, w1T, b1, w2T, b2, g, bt)


def _edge_mlp(xs, xd, ea, w1T, b1, w2T, b2):
    def k(xs_ref, xd_ref, ea_ref, w1_ref, b1_ref, w2_ref, b2_ref, o_ref):
        eav = ea_ref[...]
        z = jnp.concatenate([xs_ref[...], xd_ref[...], eav], axis=1)
        t = jnp.maximum(jnp.dot(z, w1_ref[...], preferred_element_type=F32)
                        + b1_ref[...], 0.0)
        o_ref[...] = eav + (jnp.dot(t, w2_ref[...], preferred_element_type=F32)
                            + b2_ref[...]) * 0.5
    wspec = pl.BlockSpec((3 * HP, HP), lambda i: (0, 0))
    bspec = pl.BlockSpec((1, HP), lambda i: (0, 0))
    espec = pl.BlockSpec((BE, HP), lambda i: (i, 0))
    return pl.pallas_call(
        k,
        grid=(GE,),
        in_specs=[espec, espec, espec, wspec, bspec,
                  pl.BlockSpec((HP, HP), lambda i: (0, 0)), bspec],
        out_specs=espec,
        out_shape=jax.ShapeDtypeStruct((EP, HP), F32),
        interpret=False,
    )(xs, xd, ea, w1T, b1, w2T, b2)


def _final_mlp(xs, xd, ea, w1T, b1, w2T, b2, m1T, bm1, m2T, bm2, m3, bm3):
    def k(xs_ref, xd_ref, ea_ref, w1_ref, b1_ref, w2_ref, b2_ref,
          m1_ref, bm1_ref, m2_ref, bm2_ref, m3_ref, bm3_ref, o_ref):
        xsv = xs_ref[...]
        xdv = xd_ref[...]
        eav = ea_ref[...]
        z = jnp.concatenate([xsv, xdv, eav], axis=1)
        t = jnp.maximum(jnp.dot(z, w1_ref[...], preferred_element_type=F32)
                        + b1_ref[...], 0.0)
        ea2 = eav + (jnp.dot(t, w2_ref[...], preferred_element_type=F32)
                     + b2_ref[...]) * 0.5
        z2 = jnp.concatenate([xsv, xdv, ea2], axis=1)
        o1 = jnp.maximum(jnp.dot(z2, m1_ref[...], preferred_element_type=F32)
                         + bm1_ref[...], 0.0)
        o2 = jnp.maximum(jnp.dot(o1, m2_ref[...], preferred_element_type=F32)
                         + bm2_ref[...], 0.0)
        o_ref[...] = lax.dot_general(m3_ref[...], o2, (((1,), (1,)), ((), ())),
                                     preferred_element_type=F32) + bm3_ref[...]
    wspec = pl.BlockSpec((3 * HP, HP), lambda i: (0, 0))
    hspec = pl.BlockSpec((HP, HP), lambda i: (0, 0))
    bspec = pl.BlockSpec((1, HP), lambda i: (0, 0))
    espec = pl.BlockSpec((BE, HP), lambda i: (i, 0))
    return pl.pallas_call(
        k,
        grid=(GE,),
        in_specs=[espec, espec, espec,
                  wspec, bspec, hspec, bspec,
                  wspec, bspec, hspec, bspec,
                  pl.BlockSpec((8, HP), lambda i: (0, 0)),
                  pl.BlockSpec((8, 1), lambda i: (0, 0))],
        out_specs=pl.BlockSpec((8, BE), lambda i: (0, i)),
        out_shape=jax.ShapeDtypeStruct((8, EP), F32),
        interpret=False,
    )(xs, xd, ea, w1T, b1, w2T, b2, m1T, bm1, m2T, bm2, m3, bm3)


def _pad2(w, r, c):
    return jnp.zeros((r, c), F32).at[:w.shape[0], :w.shape[1]].set(w)


def _pad_cat3(w, r):
    z = jnp.zeros((r, 3 * HP), F32)
    for t in range(3):
        z = z.at[:w.shape[0], HP * t:HP * t + H].set(w[:, H * t:H * t + H])
    return z


def _padb(b, n=HP):
    return jnp.zeros((1, n), F32).at[0, :b.shape[0]].set(b)


def kernel(x, edge_index, edge_attr, W_node, b_node, W_edge, b_edge,
           W1, b1, W2, b2, We1, be1, We2, be2, gamma, beta,
           Wm1, bm1, Wm2, bm2, Wm3, bm3):
    pad = EP - E
    padi = jnp.arange(pad, dtype=jnp.int32)
    src_flat = jnp.concatenate([edge_index[0], padi % N])
    dst_flat = jnp.concatenate([edge_index[1], N + (padi % (NP - N))])
    dst_gf = jnp.concatenate([edge_index[1], padi % N])
    ea_t = jnp.zeros((DE, EP), F32).at[:, :E].set(edge_attr.T)

    wnT = _pad2(W_node, HP, DF).T
    weT = _pad2(W_edge, HP, DE).T
    bnp = _padb(b_node)
    bep = _padb(b_edge)

    x0 = _proj_node(x, wnT, bnp)
    ea0 = _proj_edge(ea_t, weT, bep)

    xc = x0
    eac = ea0
    xs = xd = None
    for i in range(2):
        w1T = _pad2(W1[i], HP, HP).T
        w2T = _pad2(W2[i], HP, HP).T
        b1p = _padb(b1[i])
        b2p = _padb(b2[i])
        gp = _padb(gamma[i])
        btp = _padb(beta[i])
        if i == 0:
            agg = _sc_msg_scatter(xc, eac, src_flat, dst_flat, gather_x=True)
        else:
            agg = _sc_msg_scatter(xs, eac, src_flat, dst_flat, gather_x=False)
        xc = _node_mlp(xc, agg, w1T, b1p, w2T, b2p, gp, btp)
        xs, xd = _sc_gather2(xc, src_flat, dst_gf)
        if i == 0:
            we1T = _pad_cat3(We1[i], HP).T
            we2T = _pad2(We2[i], HP, HP).T
            eac = _edge_mlp(xs, xd, eac, we1T, _padb(be1[i]),
                            we2T, _padb(be2[i]))

    m1T = _pad_cat3(Wm1, HP).T
    m2T = _pad2(Wm2, HP, HP).T
    m3 = _pad2(Wm3, 8, HP)
    bm3p = jnp.zeros((8, 1), F32).at[:2, 0].set(bm3)

    we1T = _pad_cat3(We1[1], HP).T
    we2T = _pad2(We2[1], HP, HP).T

    outT = _final_mlp(xs, xd, eac, we1T, _padb(be1[1]),
                      we2T, _padb(be2[1]),
                      m1T, _padb(bm1), m2T, _padb(bm2), m3, bm3p)
    return outT[:2, :E].T


# speedup vs baseline: 1.1296x; 1.0056x over previous
import functools

import jax
import jax.numpy as jnp
from jax import lax
from jax.experimental import pallas as pl
from jax.experimental.pallas import tpu as pltpu
from jax.experimental.pallas import tpu_sc as plsc

F32 = jnp.float32

N = 10000
E = 320000
DF = 128
DE = 16
H = 100
HP = 128
NC = 2
NS = 16
NW = NC * NS
EP = 327680
EPW = EP // NW
CH = 128
NCHUNK = EPW // CH
CH_MS = 64
NCH_MS = EPW // CH_MS
NP = 10240
RPS = NP // NS
BE = 2048
GE = EP // BE

_mesh = plsc.VectorSubcoreMesh(core_axis_name="c", subcore_axis_name="s")


def _zero_vmem_rows(buf, rows):
    def body(i, _):
        for j in range(HP // 16):
            buf[i, pl.ds(j * 16, 16)] = jnp.zeros((16,), F32)
        return 0
    lax.fori_loop(0, rows, body, 0)


def _relu_add_rows(dst_buf, a_buf, rows):
    def body(i, _):
        for j in range(HP // 16):
            sl = pl.ds(j * 16, 16)
            dst_buf[i, sl] = jnp.maximum(a_buf[i, sl] + dst_buf[i, sl], 0.0)
        return 0
    lax.fori_loop(0, rows, body, 0)


def _sc_msg_scatter(x0, ea0, src_p, dst_p, gather_x):
    @functools.partial(
        pl.kernel,
        out_type=jax.ShapeDtypeStruct((NC * NP, HP), F32),
        mesh=_mesh,
        scratch_types=[
            pltpu.VMEM((1, CH_MS), jnp.int32),
            pltpu.VMEM((1, CH_MS), jnp.int32),
            pltpu.VMEM((1, CH_MS), jnp.int32),
            pltpu.VMEM((1, CH_MS), jnp.int32),
            pltpu.VMEM((CH_MS, HP), F32),
            pltpu.VMEM((CH_MS, HP), F32),
            pltpu.VMEM((CH_MS, HP), F32),
            pltpu.VMEM((CH_MS, HP), F32),
            pltpu.SemaphoreType.DMA,
            pltpu.SemaphoreType.DMA,
            pltpu.SemaphoreType.DMA,
            pltpu.SemaphoreType.DMA,
            pltpu.SemaphoreType.DMA,
            pltpu.SemaphoreType.DMA,
            pltpu.SemaphoreType.DMA,
            pltpu.SemaphoreType.DMA,
            pltpu.VMEM_SHARED((NP, HP), F32),
        ],
        interpret=False,
    )
    def k(x_hbm, ea_hbm, src_hbm, dst_hbm, out_hbm,
          is0, is1, id0, id1, xr0, xr1, eb0, eb1,
          sg0, sg1, se0, se1, ss0, ss1, sd0, sd1, agg_sh):
        c = lax.axis_index("c")
        s = lax.axis_index("s")
        gwid = c * NS + s
        isb = (is0, is1)
        idb = (id0, id1)
        xr = (xr0, xr1)
        eb = (eb0, eb1)
        sg = (sg0, sg1)
        se = (se0, se1)
        ss = (ss0, ss1)
        sd = (sd0, sd1)

        _zero_vmem_rows(xr0, CH_MS)
        def zc(t, _):
            pltpu.sync_copy(xr0, agg_sh.at[pl.ds(s * RPS + t * CH_MS,
                                                 CH_MS)])
            return 0
        lax.fori_loop(0, RPS // CH_MS, zc, 0)
        plsc.subcore_barrier()

        def start_idx(kk, b):
            base = gwid * EPW + kk * CH_MS
            pltpu.async_copy(dst_hbm.at[pl.ds(base, CH_MS)], idb[b].at[0],
                             sd[b])
            if gather_x:
                pltpu.async_copy(src_hbm.at[pl.ds(base, CH_MS)],
                                 isb[b].at[0], ss[b])

        def wait_idx(kk, b):
            base = gwid * EPW + kk * CH_MS
            pltpu.make_async_copy(dst_hbm.at[pl.ds(base, CH_MS)],
                                  idb[b].at[0], sd[b]).wait()
            if gather_x:
                pltpu.make_async_copy(src_hbm.at[pl.ds(base, CH_MS)],
                                      isb[b].at[0], ss[b]).wait()

        def start_in(kk, b):
            base = gwid * EPW + kk * CH_MS
            if gather_x:
                pltpu.async_copy(x_hbm.at[isb[b].at[0]], xr[b], sg[b])
            else:
                pltpu.async_copy(x_hbm.at[pl.ds(base, CH_MS)], xr[b], sg[b])
            pltpu.async_copy(ea_hbm.at[pl.ds(base, CH_MS)], eb[b], se[b])

        def wait_in(kk, b):
            base = gwid * EPW + kk * CH_MS
            if gather_x:
                pltpu.make_async_copy(x_hbm.at[isb[b].at[0]], xr[b],
                                      sg[b]).wait()
            else:
                pltpu.make_async_copy(x_hbm.at[pl.ds(base, CH_MS)], xr[b],
                                      sg[b]).wait()
            pltpu.make_async_copy(ea_hbm.at[pl.ds(base, CH_MS)], eb[b],
                                  se[b]).wait()

        start_idx(0, 0)
        wait_idx(0, 0)
        start_in(0, 0)
        start_idx(1, 1)

        def body(t, _):
            for b in range(2):
                kk = t * 2 + b
                nb = 1 - b
                @pl.when(kk + 1 < NCH_MS)
                def _():
                    wait_idx(kk + 1, nb)
                    start_in(kk + 1, nb)
                wait_in(kk, b)
                _relu_add_rows(eb[b], xr[b], CH_MS)
                pltpu.sync_copy(eb[b], agg_sh.at[idb[b].at[0]], add=True)
                @pl.when(kk + 2 < NCH_MS)
                def _():
                    start_idx(kk + 2, b)
            return 0
        lax.fori_loop(0, NCH_MS // 2, body, 0)
        plsc.subcore_barrier()
        pltpu.sync_copy(agg_sh.at[pl.ds(s * RPS, RPS)],
                        out_hbm.at[pl.ds(c * NP + s * RPS, RPS)])

    return k(x0, ea0, src_p, dst_p)


def _sc_gather2(x0, src_p, dst_p, off=0, rows=EP, dep=None):
    epw = rows // NW
    nch = epw // CH_MS
    if dep is None:
        dep = x0

    @functools.partial(
        pl.kernel,
        out_type=(jax.ShapeDtypeStruct((rows, HP), F32),
                  jax.ShapeDtypeStruct((rows, HP), F32)),
        mesh=_mesh,
        scratch_types=[
            pltpu.VMEM((1, CH_MS), jnp.int32),
            pltpu.VMEM((1, CH_MS), jnp.int32),
            pltpu.VMEM((1, CH_MS), jnp.int32),
            pltpu.VMEM((1, CH_MS), jnp.int32),
            pltpu.VMEM((CH_MS, HP), F32),
            pltpu.VMEM((CH_MS, HP), F32),
            pltpu.VMEM((CH_MS, HP), F32),
            pltpu.VMEM((CH_MS, HP), F32),
            pltpu.SemaphoreType.DMA,
            pltpu.SemaphoreType.DMA,
            pltpu.SemaphoreType.DMA,
            pltpu.SemaphoreType.DMA,
            pltpu.SemaphoreType.DMA,
            pltpu.SemaphoreType.DMA,
            pltpu.SemaphoreType.DMA,
            pltpu.SemaphoreType.DMA,
            pltpu.SemaphoreType.DMA,
            pltpu.SemaphoreType.DMA,
            pltpu.SemaphoreType.DMA,
            pltpu.SemaphoreType.DMA,
            pltpu.VMEM_SHARED((N, HP), F32),
        ],
        interpret=False,
    )
    def k(x_hbm, src_hbm, dst_hbm, dep_hbm, xs_hbm, xd_hbm,
          is0, is1, id0, id1, bs0, bs1, bd0, bd1,
          gs0, gs1, gd0, gd1, ws0, ws1, wd0, wd1,
          ss0, ss1, sd0, sd1, x_sh):
        c = lax.axis_index("c")
        s = lax.axis_index("s")
        gwid = c * NS + s
        isb = (is0, is1)
        idb = (id0, id1)
        bs = (bs0, bs1)
        bd = (bd0, bd1)
        gs = (gs0, gs1)
        gd = (gd0, gd1)
        ws = (ws0, ws1)
        wd = (wd0, wd1)
        ss = (ss0, ss1)
        sd = (sd0, sd1)

        pltpu.sync_copy(x_hbm.at[pl.ds(s * 624, 640)],
                        x_sh.at[pl.ds(s * 624, 640)])
        plsc.subcore_barrier()

        def start_idx(kk, b):
            base = off + gwid * epw + kk * CH_MS
            pltpu.async_copy(src_hbm.at[pl.ds(base, CH_MS)], isb[b].at[0],
                             ss[b])
            pltpu.async_copy(dst_hbm.at[pl.ds(base, CH_MS)], idb[b].at[0],
                             sd[b])

        def wait_idx(kk, b):
            base = off + gwid * epw + kk * CH_MS
            pltpu.make_async_copy(src_hbm.at[pl.ds(base, CH_MS)],
                                  isb[b].at[0], ss[b]).wait()
            pltpu.make_async_copy(dst_hbm.at[pl.ds(base, CH_MS)],
                                  idb[b].at[0], sd[b]).wait()

        def start_in(kk, b):
            pltpu.async_copy(x_sh.at[isb[b].at[0]], bs[b], gs[b])
            pltpu.async_copy(x_sh.at[idb[b].at[0]], bd[b], gd[b])

        def wait_in(kk, b):
            pltpu.make_async_copy(x_sh.at[isb[b].at[0]], bs[b], gs[b]).wait()
            pltpu.make_async_copy(x_sh.at[idb[b].at[0]], bd[b], gd[b]).wait()

        def wait_out(kk, b):
            base = gwid * epw + kk * CH_MS
            pltpu.make_async_copy(bs[b], xs_hbm.at[pl.ds(base, CH_MS)],
                                  ws[b]).wait()
            pltpu.make_async_copy(bd[b], xd_hbm.at[pl.ds(base, CH_MS)],
                                  wd[b]).wait()

        start_idx(0, 0)
        wait_idx(0, 0)
        start_in(0, 0)
        start_idx(1, 1)

        def body(t, _):
            for b in range(2):
                kk = t * 2 + b
                nb = 1 - b
                @pl.when(kk + 1 < nch)
                def _():
                    wait_idx(kk + 1, nb)
                    @pl.when(kk >= 1)
                    def _():
                        wait_out(kk - 1, nb)
                    start_in(kk + 1, nb)
                wait_in(kk, b)
                @pl.when(kk + 2 < nch)
                def _():
                    start_idx(kk + 2, b)
                base = gwid * epw + kk * CH_MS
                pltpu.async_copy(bs[b], xs_hbm.at[pl.ds(base, CH_MS)], ws[b])
                pltpu.async_copy(bd[b], xd_hbm.at[pl.ds(base, CH_MS)], wd[b])
            return 0
        lax.fori_loop(0, nch // 2, body, 0)
        wait_out(nch - 2, 0)
        wait_out(nch - 1, 1)

    return k(x0, src_p, dst_p, dep)


def _proj_node(x, wT, b):
    def k(x_ref, w_ref, b_ref, o_ref):
        o_ref[...] = jnp.dot(x_ref[...], w_ref[...],
                             preferred_element_type=F32) + b_ref[...]
    return pl.pallas_call(
        k,
        out_shape=jax.ShapeDtypeStruct((N, HP), F32),
        interpret=False,
    )(x, wT, b)


def _proj_edge(ea_t, wT, b):
    def k(a_ref, w_ref, b_ref, o_ref):
        o_ref[...] = lax.dot_general(
            a_ref[...], w_ref[...], (((0,), (0,)), ((), ())),
            preferred_element_type=F32) + b_ref[...]
    return pl.pallas_call(
        k,
        grid=(GE,),
        in_specs=[
            pl.BlockSpec((DE, BE), lambda i: (0, i)),
            pl.BlockSpec((DE, HP), lambda i: (0, 0)),
            pl.BlockSpec((1, HP), lambda i: (0, 0)),
        ],
        out_specs=pl.BlockSpec((BE, HP), lambda i: (i, 0)),
        out_shape=jax.ShapeDtypeStruct((EP, HP), F32),
        interpret=False,
    )(ea_t, wT, b)


def _node_mlp(x, agg, w1T, b1, w2T, b2, g, bt):
    def k(x_ref, a_ref, w1_ref, b1_ref, w2_ref, b2_ref, g_ref, bt_ref, o_ref):
        xv = x_ref[...]
        a = a_ref[0:N, :] + a_ref[NP:NP + N, :]
        h = xv + a
        h = jnp.maximum(jnp.dot(h, w1_ref[...], preferred_element_type=F32)
                        + b1_ref[...], 0.0)
        h = jnp.dot(h, w2_ref[...], preferred_element_type=F32) + b2_ref[...]
        m = jnp.mean(h, axis=0, keepdims=True)
        v = jnp.mean((h - m) ** 2, axis=0, keepdims=True)
        hn = (h - m) * lax.rsqrt(v + 1e-5) * g_ref[...] + bt_ref[...]
        o_ref[...] = (xv + jnp.maximum(hn, 0.0)) * 0.5
    return pl.pallas_call(
        k,
        out_shape=jax.ShapeDtypeStruct((N, HP), F32),
        interpret=False,
    )(x, agg, w1T, b1, w2T, b2, g, bt)


def _edge_mlp(xs, xd, ea, w1T, b1, w2T, b2):
    def k(xs_ref, xd_ref, ea_ref, w1_ref, b1_ref, w2_ref, b2_ref, o_ref):
        eav = ea_ref[...]
        z = jnp.concatenate([xs_ref[...], xd_ref[...], eav], axis=1)
        t = jnp.maximum(jnp.dot(z, w1_ref[...], preferred_element_type=F32)
                        + b1_ref[...], 0.0)
        o_ref[...] = eav + (jnp.dot(t, w2_ref[...], preferred_element_type=F32)
                            + b2_ref[...]) * 0.5
    wspec = pl.BlockSpec((3 * HP, HP), lambda i: (0, 0))
    bspec = pl.BlockSpec((1, HP), lambda i: (0, 0))
    espec = pl.BlockSpec((BE, HP), lambda i: (i, 0))
    return pl.pallas_call(
        k,
        grid=(GE,),
        in_specs=[espec, espec, espec, wspec, bspec,
                  pl.BlockSpec((HP, HP), lambda i: (0, 0)), bspec],
        out_specs=espec,
        out_shape=jax.ShapeDtypeStruct((EP, HP), F32),
        interpret=False,
    )(xs, xd, ea, w1T, b1, w2T, b2)


def _final_mlp(xs, xd, ea, w1T, b1, w2T, b2, m1T, bm1, m2T, bm2, m3, bm3,
               hoff=0, rows=EP):
    def k(xs_ref, xd_ref, ea_ref, w1_ref, b1_ref, w2_ref, b2_ref,
          m1_ref, bm1_ref, m2_ref, bm2_ref, m3_ref, bm3_ref, o_ref):
        xsv = xs_ref[...]
        xdv = xd_ref[...]
        eav = ea_ref[...]
        z = jnp.concatenate([xsv, xdv, eav], axis=1)
        t = jnp.maximum(jnp.dot(z, w1_ref[...], preferred_element_type=F32)
                        + b1_ref[...], 0.0)
        ea2 = eav + (jnp.dot(t, w2_ref[...], preferred_element_type=F32)
                     + b2_ref[...]) * 0.5
        z2 = jnp.concatenate([xsv, xdv, ea2], axis=1)
        o1 = jnp.maximum(jnp.dot(z2, m1_ref[...], preferred_element_type=F32)
                         + bm1_ref[...], 0.0)
        o2 = jnp.maximum(jnp.dot(o1, m2_ref[...], preferred_element_type=F32)
                         + bm2_ref[...], 0.0)
        o_ref[...] = lax.dot_general(m3_ref[...], o2, (((1,), (1,)), ((), ())),
                                     preferred_element_type=F32) + bm3_ref[...]
    goff = hoff // BE
    wspec = pl.BlockSpec((3 * HP, HP), lambda i: (0, 0))
    hspec = pl.BlockSpec((HP, HP), lambda i: (0, 0))
    bspec = pl.BlockSpec((1, HP), lambda i: (0, 0))
    espec = pl.BlockSpec((BE, HP), lambda i: (i, 0))
    easpec = pl.BlockSpec((BE, HP), lambda i: (i + goff, 0))
    return pl.pallas_call(
        k,
        grid=(rows // BE,),
        in_specs=[espec, espec, easpec,
                  wspec, bspec, hspec, bspec,
                  wspec, bspec, hspec, bspec,
                  pl.BlockSpec((8, HP), lambda i: (0, 0)),
                  pl.BlockSpec((8, 1), lambda i: (0, 0))],
        out_specs=pl.BlockSpec((8, BE), lambda i: (0, i)),
        out_shape=jax.ShapeDtypeStruct((8, rows), F32),
        interpret=False,
    )(xs, xd, ea, w1T, b1, w2T, b2, m1T, bm1, m2T, bm2, m3, bm3)


def _pad2(w, r, c):
    return jnp.zeros((r, c), F32).at[:w.shape[0], :w.shape[1]].set(w)


def _pad_cat3(w, r):
    z = jnp.zeros((r, 3 * HP), F32)
    for t in range(3):
        z = z.at[:w.shape[0], HP * t:HP * t + H].set(w[:, H * t:H * t + H])
    return z


def _padb(b, n=HP):
    return jnp.zeros((1, n), F32).at[0, :b.shape[0]].set(b)


def kernel(x, edge_index, edge_attr, W_node, b_node, W_edge, b_edge,
           W1, b1, W2, b2, We1, be1, We2, be2, gamma, beta,
           Wm1, bm1, Wm2, bm2, Wm3, bm3):
    pad = EP - E
    padi = jnp.arange(pad, dtype=jnp.int32)
    src_flat = jnp.concatenate([edge_index[0], padi % N])
    dst_flat = jnp.concatenate([edge_index[1], N + (padi % (NP - N))])
    dst_gf = jnp.concatenate([edge_index[1], padi % N])
    ea_t = jnp.zeros((DE, EP), F32).at[:, :E].set(edge_attr.T)

    wnT = _pad2(W_node, HP, DF).T
    weT = _pad2(W_edge, HP, DE).T
    bnp = _padb(b_node)
    bep = _padb(b_edge)

    x0 = _proj_node(x, wnT, bnp)
    ea0 = _proj_edge(ea_t, weT, bep)

    xc = x0
    eac = ea0
    xs = xd = None
    for i in range(2):
        w1T = _pad2(W1[i], HP, HP).T
        w2T = _pad2(W2[i], HP, HP).T
        b1p = _padb(b1[i])
        b2p = _padb(b2[i])
        gp = _padb(gamma[i])
        btp = _padb(beta[i])
        if i == 0:
            agg = _sc_msg_scatter(xc, eac, src_flat, dst_flat, gather_x=True)
        else:
            agg = _sc_msg_scatter(xs, eac, src_flat, dst_flat, gather_x=False)
        xc = _node_mlp(xc, agg, w1T, b1p, w2T, b2p, gp, btp)
        if i == 0:
            xs, xd = _sc_gather2(xc, src_flat, dst_gf)
            we1T = _pad_cat3(We1[i], HP).T
            we2T = _pad2(We2[i], HP, HP).T
            eac = _edge_mlp(xs, xd, eac, we1T, _padb(be1[i]),
                            we2T, _padb(be2[i]))

    m1T = _pad_cat3(Wm1, HP).T
    m2T = _pad2(Wm2, HP, HP).T
    m3 = _pad2(Wm3, 8, HP)
    bm3p = jnp.zeros((8, 1), F32).at[:2, 0].set(bm3)

    we1T = _pad_cat3(We1[1], HP).T
    we2T = _pad2(We2[1], HP, HP).T

    EPH = EP // 2
    xsA, xdA = _sc_gather2(xc, src_flat, dst_gf, off=0, rows=EPH)
    xsB, xdB = _sc_gather2(xc, src_flat, dst_gf, off=EPH, rows=EPH, dep=xsA)
    outA = _final_mlp(xsA, xdA, eac, we1T, _padb(be1[1]),
                      we2T, _padb(be2[1]),
                      m1T, _padb(bm1), m2T, _padb(bm2), m3, bm3p,
                      hoff=0, rows=EPH)
    outB = _final_mlp(xsB, xdB, eac, we1T, _padb(be1[1]),
                      we2T, _padb(be2[1]),
                      m1T, _padb(bm1), m2T, _padb(bm2), m3, bm3p,
                      hoff=EPH, rows=EPH)
    outT = jnp.concatenate([outA, outB], axis=1)
    return outT[:2, :E].T


# speedup vs baseline: 1.2232x; 1.0829x over previous
import functools

import jax
import jax.numpy as jnp
from jax import lax
from jax.experimental import pallas as pl
from jax.experimental.pallas import tpu as pltpu
from jax.experimental.pallas import tpu_sc as plsc

F32 = jnp.float32

N = 10000
E = 320000
DF = 128
DE = 16
H = 100
HP = 128
NC = 2
NS = 16
NW = NC * NS
EP = 327680
EPW = EP // NW
CH = 128
NCHUNK = EPW // CH
CH_MS = 64
NCH_MS = EPW // CH_MS
NP = 10240
RPS = NP // NS
BE = 2048
GE = EP // BE

_mesh = plsc.VectorSubcoreMesh(core_axis_name="c", subcore_axis_name="s")


def _zero_vmem_rows(buf, rows):
    def body(i, _):
        for j in range(HP // 16):
            buf[i, pl.ds(j * 16, 16)] = jnp.zeros((16,), F32)
        return 0
    lax.fori_loop(0, rows, body, 0)


def _relu_add_rows(dst_buf, a_buf, rows):
    def body(i, _):
        for j in range(HP // 16):
            sl = pl.ds(j * 16, 16)
            dst_buf[i, sl] = jnp.maximum(a_buf[i, sl] + dst_buf[i, sl], 0.0)
        return 0
    lax.fori_loop(0, rows, body, 0)


def _sc_msg_scatter(x0, ea0, src_p, dst_p, gather_x,
                    xoff=0, eaoff=0, idxoff=0, rows=EP, dep=None):
    epw = rows // NW
    nch = epw // CH_MS
    if dep is None:
        dep = ea0

    @functools.partial(
        pl.kernel,
        out_type=jax.ShapeDtypeStruct((NC * NP, HP), F32),
        mesh=_mesh,
        scratch_types=[
            pltpu.VMEM((1, CH_MS), jnp.int32),
            pltpu.VMEM((1, CH_MS), jnp.int32),
            pltpu.VMEM((1, CH_MS), jnp.int32),
            pltpu.VMEM((1, CH_MS), jnp.int32),
            pltpu.VMEM((CH_MS, HP), F32),
            pltpu.VMEM((CH_MS, HP), F32),
            pltpu.VMEM((CH_MS, HP), F32),
            pltpu.VMEM((CH_MS, HP), F32),
            pltpu.SemaphoreType.DMA,
            pltpu.SemaphoreType.DMA,
            pltpu.SemaphoreType.DMA,
            pltpu.SemaphoreType.DMA,
            pltpu.SemaphoreType.DMA,
            pltpu.SemaphoreType.DMA,
            pltpu.SemaphoreType.DMA,
            pltpu.SemaphoreType.DMA,
            pltpu.VMEM_SHARED((NP, HP), F32),
        ],
        interpret=False,
    )
    def k(x_hbm, ea_hbm, src_hbm, dst_hbm, dep_hbm, out_hbm,
          is0, is1, id0, id1, xr0, xr1, eb0, eb1,
          sg0, sg1, se0, se1, ss0, ss1, sd0, sd1, agg_sh):
        c = lax.axis_index("c")
        s = lax.axis_index("s")
        gwid = c * NS + s
        isb = (is0, is1)
        idb = (id0, id1)
        xr = (xr0, xr1)
        eb = (eb0, eb1)
        sg = (sg0, sg1)
        se = (se0, se1)
        ss = (ss0, ss1)
        sd = (sd0, sd1)

        _zero_vmem_rows(xr0, CH_MS)
        def zc(t, _):
            pltpu.sync_copy(xr0, agg_sh.at[pl.ds(s * RPS + t * CH_MS,
                                                 CH_MS)])
            return 0
        lax.fori_loop(0, RPS // CH_MS, zc, 0)
        plsc.subcore_barrier()

        def start_idx(kk, b):
            ibase = idxoff + gwid * epw + kk * CH_MS
            ebase = eaoff + gwid * epw + kk * CH_MS
            xbase = xoff + gwid * epw + kk * CH_MS
            pltpu.async_copy(dst_hbm.at[pl.ds(ibase, CH_MS)], idb[b].at[0],
                             sd[b])
            if gather_x:
                pltpu.async_copy(src_hbm.at[pl.ds(ibase, CH_MS)],
                                 isb[b].at[0], ss[b])

        def wait_idx(kk, b):
            ibase = idxoff + gwid * epw + kk * CH_MS
            ebase = eaoff + gwid * epw + kk * CH_MS
            xbase = xoff + gwid * epw + kk * CH_MS
            pltpu.make_async_copy(dst_hbm.at[pl.ds(ibase, CH_MS)],
                                  idb[b].at[0], sd[b]).wait()
            if gather_x:
                pltpu.make_async_copy(src_hbm.at[pl.ds(ibase, CH_MS)],
                                      isb[b].at[0], ss[b]).wait()

        def start_in(kk, b):
            ibase = idxoff + gwid * epw + kk * CH_MS
            ebase = eaoff + gwid * epw + kk * CH_MS
            xbase = xoff + gwid * epw + kk * CH_MS
            if gather_x:
                pltpu.async_copy(x_hbm.at[isb[b].at[0]], xr[b], sg[b])
            else:
                pltpu.async_copy(x_hbm.at[pl.ds(xbase, CH_MS)], xr[b], sg[b])
            pltpu.async_copy(ea_hbm.at[pl.ds(ebase, CH_MS)], eb[b], se[b])

        def wait_in(kk, b):
            ibase = idxoff + gwid * epw + kk * CH_MS
            ebase = eaoff + gwid * epw + kk * CH_MS
            xbase = xoff + gwid * epw + kk * CH_MS
            if gather_x:
                pltpu.make_async_copy(x_hbm.at[isb[b].at[0]], xr[b],
                                      sg[b]).wait()
            else:
                pltpu.make_async_copy(x_hbm.at[pl.ds(xbase, CH_MS)], xr[b],
                                      sg[b]).wait()
            pltpu.make_async_copy(ea_hbm.at[pl.ds(ebase, CH_MS)], eb[b],
                                  se[b]).wait()

        start_idx(0, 0)
        wait_idx(0, 0)
        start_in(0, 0)
        start_idx(1, 1)

        def body(t, _):
            for b in range(2):
                kk = t * 2 + b
                nb = 1 - b
                @pl.when(kk + 1 < nch)
                def _():
                    wait_idx(kk + 1, nb)
                    start_in(kk + 1, nb)
                wait_in(kk, b)
                _relu_add_rows(eb[b], xr[b], CH_MS)
                pltpu.sync_copy(eb[b], agg_sh.at[idb[b].at[0]], add=True)
                @pl.when(kk + 2 < nch)
                def _():
                    start_idx(kk + 2, b)
            return 0
        lax.fori_loop(0, nch // 2, body, 0)
        plsc.subcore_barrier()
        pltpu.sync_copy(agg_sh.at[pl.ds(s * RPS, RPS)],
                        out_hbm.at[pl.ds(c * NP + s * RPS, RPS)])

    return k(x0, ea0, src_p, dst_p, dep)


def _sc_gather2(x0, src_p, dst_p, off=0, rows=EP, dep=None):
    epw = rows // NW
    nch = epw // CH_MS
    if dep is None:
        dep = x0

    @functools.partial(
        pl.kernel,
        out_type=(jax.ShapeDtypeStruct((rows, HP), F32),
                  jax.ShapeDtypeStruct((rows, HP), F32)),
        mesh=_mesh,
        scratch_types=[
            pltpu.VMEM((1, CH_MS), jnp.int32),
            pltpu.VMEM((1, CH_MS), jnp.int32),
            pltpu.VMEM((1, CH_MS), jnp.int32),
            pltpu.VMEM((1, CH_MS), jnp.int32),
            pltpu.VMEM((CH_MS, HP), F32),
            pltpu.VMEM((CH_MS, HP), F32),
            pltpu.VMEM((CH_MS, HP), F32),
            pltpu.VMEM((CH_MS, HP), F32),
            pltpu.SemaphoreType.DMA,
            pltpu.SemaphoreType.DMA,
            pltpu.SemaphoreType.DMA,
            pltpu.SemaphoreType.DMA,
            pltpu.SemaphoreType.DMA,
            pltpu.SemaphoreType.DMA,
            pltpu.SemaphoreType.DMA,
            pltpu.SemaphoreType.DMA,
            pltpu.SemaphoreType.DMA,
            pltpu.SemaphoreType.DMA,
            pltpu.SemaphoreType.DMA,
            pltpu.SemaphoreType.DMA,
            pltpu.VMEM_SHARED((N, HP), F32),
        ],
        interpret=False,
    )
    def k(x_hbm, src_hbm, dst_hbm, dep_hbm, xs_hbm, xd_hbm,
          is0, is1, id0, id1, bs0, bs1, bd0, bd1,
          gs0, gs1, gd0, gd1, ws0, ws1, wd0, wd1,
          ss0, ss1, sd0, sd1, x_sh):
        c = lax.axis_index("c")
        s = lax.axis_index("s")
        gwid = c * NS + s
        isb = (is0, is1)
        idb = (id0, id1)
        bs = (bs0, bs1)
        bd = (bd0, bd1)
        gs = (gs0, gs1)
        gd = (gd0, gd1)
        ws = (ws0, ws1)
        wd = (wd0, wd1)
        ss = (ss0, ss1)
        sd = (sd0, sd1)

        pltpu.sync_copy(x_hbm.at[pl.ds(s * 624, 640)],
                        x_sh.at[pl.ds(s * 624, 640)])
        plsc.subcore_barrier()

        def start_idx(kk, b):
            base = off + gwid * epw + kk * CH_MS
            pltpu.async_copy(src_hbm.at[pl.ds(base, CH_MS)], isb[b].at[0],
                             ss[b])
            pltpu.async_copy(dst_hbm.at[pl.ds(base, CH_MS)], idb[b].at[0],
                             sd[b])

        def wait_idx(kk, b):
            base = off + gwid * epw + kk * CH_MS
            pltpu.make_async_copy(src_hbm.at[pl.ds(base, CH_MS)],
                                  isb[b].at[0], ss[b]).wait()
            pltpu.make_async_copy(dst_hbm.at[pl.ds(base, CH_MS)],
                                  idb[b].at[0], sd[b]).wait()

        def start_in(kk, b):
            pltpu.async_copy(x_sh.at[isb[b].at[0]], bs[b], gs[b])
            pltpu.async_copy(x_sh.at[idb[b].at[0]], bd[b], gd[b])

        def wait_in(kk, b):
            pltpu.make_async_copy(x_sh.at[isb[b].at[0]], bs[b], gs[b]).wait()
            pltpu.make_async_copy(x_sh.at[idb[b].at[0]], bd[b], gd[b]).wait()

        def wait_out(kk, b):
            base = gwid * epw + kk * CH_MS
            pltpu.make_async_copy(bs[b], xs_hbm.at[pl.ds(base, CH_MS)],
                                  ws[b]).wait()
            pltpu.make_async_copy(bd[b], xd_hbm.at[pl.ds(base, CH_MS)],
                                  wd[b]).wait()

        start_idx(0, 0)
        wait_idx(0, 0)
        start_in(0, 0)
        start_idx(1, 1)

        def body(t, _):
            for b in range(2):
                kk = t * 2 + b
                nb = 1 - b
                @pl.when(kk + 1 < nch)
                def _():
                    wait_idx(kk + 1, nb)
                    @pl.when(kk >= 1)
                    def _():
                        wait_out(kk - 1, nb)
                    start_in(kk + 1, nb)
                wait_in(kk, b)
                @pl.when(kk + 2 < nch)
                def _():
                    start_idx(kk + 2, b)
                base = gwid * epw + kk * CH_MS
                pltpu.async_copy(bs[b], xs_hbm.at[pl.ds(base, CH_MS)], ws[b])
                pltpu.async_copy(bd[b], xd_hbm.at[pl.ds(base, CH_MS)], wd[b])
            return 0
        lax.fori_loop(0, nch // 2, body, 0)
        wait_out(nch - 2, 0)
        wait_out(nch - 1, 1)

    return k(x0, src_p, dst_p, dep)


def _proj_node(x, wT, b):
    def k(x_ref, w_ref, b_ref, o_ref):
        o_ref[...] = jnp.dot(x_ref[...], w_ref[...],
                             preferred_element_type=F32) + b_ref[...]
    return pl.pallas_call(
        k,
        out_shape=jax.ShapeDtypeStruct((N, HP), F32),
        interpret=False,
    )(x, wT, b)


def _proj_edge(ea_t, wT, b, hoff=0, rows=EP):
    goff = hoff // BE
    def k(a_ref, w_ref, b_ref, o_ref):
        o_ref[...] = lax.dot_general(
            a_ref[...], w_ref[...], (((0,), (0,)), ((), ())),
            preferred_element_type=F32) + b_ref[...]
    return pl.pallas_call(
        k,
        grid=(rows // BE,),
        in_specs=[
            pl.BlockSpec((DE, BE), lambda i: (0, i + goff)),
            pl.BlockSpec((DE, HP), lambda i: (0, 0)),
            pl.BlockSpec((1, HP), lambda i: (0, 0)),
        ],
        out_specs=pl.BlockSpec((BE, HP), lambda i: (i, 0)),
        out_shape=jax.ShapeDtypeStruct((rows, HP), F32),
        interpret=False,
    )(ea_t, wT, b)


def _node_mlp(x, agg, agg2, w1T, b1, w2T, b2, g, bt):
    def k(x_ref, a_ref, a2_ref, w1_ref, b1_ref, w2_ref, b2_ref, g_ref,
          bt_ref, o_ref):
        xv = x_ref[...]
        a = (a_ref[0:N, :] + a_ref[NP:NP + N, :]
             + a2_ref[0:N, :] + a2_ref[NP:NP + N, :])
        h = xv + a
        h = jnp.maximum(jnp.dot(h, w1_ref[...], preferred_element_type=F32)
                        + b1_ref[...], 0.0)
        h = jnp.dot(h, w2_ref[...], preferred_element_type=F32) + b2_ref[...]
        m = jnp.mean(h, axis=0, keepdims=True)
        v = jnp.mean((h - m) ** 2, axis=0, keepdims=True)
        hn = (h - m) * lax.rsqrt(v + 1e-5) * g_ref[...] + bt_ref[...]
        o_ref[...] = (xv + jnp.maximum(hn, 0.0)) * 0.5
    return pl.pallas_call(
        k,
        out_shape=jax.ShapeDtypeStruct((N, HP), F32),
        interpret=False,
    )(x, agg, agg2, w1T, b1, w2T, b2, g, bt)


def _edge_mlp(xs, xd, ea, w1T, b1, w2T, b2):
    def k(xs_ref, xd_ref, ea_ref, w1_ref, b1_ref, w2_ref, b2_ref, o_ref):
        eav = ea_ref[...]
        z = jnp.concatenate([xs_ref[...], xd_ref[...], eav], axis=1)
        t = jnp.maximum(jnp.dot(z, w1_ref[...], preferred_element_type=F32)
                        + b1_ref[...], 0.0)
        o_ref[...] = eav + (jnp.dot(t, w2_ref[...], preferred_element_type=F32)
                            + b2_ref[...]) * 0.5
    wspec = pl.BlockSpec((3 * HP, HP), lambda i: (0, 0))
    bspec = pl.BlockSpec((1, HP), lambda i: (0, 0))
    espec = pl.BlockSpec((BE, HP), lambda i: (i, 0))
    rows = xs.shape[0]
    return pl.pallas_call(
        k,
        grid=(rows // BE,),
        in_specs=[espec, espec, espec, wspec, bspec,
                  pl.BlockSpec((HP, HP), lambda i: (0, 0)), bspec],
        out_specs=espec,
        out_shape=jax.ShapeDtypeStruct((rows, HP), F32),
        interpret=False,
    )(xs, xd, ea, w1T, b1, w2T, b2)


def _final_mlp(xs, xd, ea, w1T, b1, w2T, b2, m1T, bm1, m2T, bm2, m3, bm3,
               rows=EP):
    def k(xs_ref, xd_ref, ea_ref, w1_ref, b1_ref, w2_ref, b2_ref,
          m1_ref, bm1_ref, m2_ref, bm2_ref, m3_ref, bm3_ref, o_ref):
        xsv = xs_ref[...]
        xdv = xd_ref[...]
        eav = ea_ref[...]
        z = jnp.concatenate([xsv, xdv, eav], axis=1)
        t = jnp.maximum(jnp.dot(z, w1_ref[...], preferred_element_type=F32)
                        + b1_ref[...], 0.0)
        ea2 = eav + (jnp.dot(t, w2_ref[...], preferred_element_type=F32)
                     + b2_ref[...]) * 0.5
        z2 = jnp.concatenate([xsv, xdv, ea2], axis=1)
        o1 = jnp.maximum(jnp.dot(z2, m1_ref[...], preferred_element_type=F32)
                         + bm1_ref[...], 0.0)
        o2 = jnp.maximum(jnp.dot(o1, m2_ref[...], preferred_element_type=F32)
                         + bm2_ref[...], 0.0)
        o_ref[...] = lax.dot_general(m3_ref[...], o2, (((1,), (1,)), ((), ())),
                                     preferred_element_type=F32) + bm3_ref[...]
    wspec = pl.BlockSpec((3 * HP, HP), lambda i: (0, 0))
    hspec = pl.BlockSpec((HP, HP), lambda i: (0, 0))
    bspec = pl.BlockSpec((1, HP), lambda i: (0, 0))
    espec = pl.BlockSpec((BE, HP), lambda i: (i, 0))
    return pl.pallas_call(
        k,
        grid=(rows // BE,),
        in_specs=[espec, espec, espec,
                  wspec, bspec, hspec, bspec,
                  wspec, bspec, hspec, bspec,
                  pl.BlockSpec((8, HP), lambda i: (0, 0)),
                  pl.BlockSpec((8, 1), lambda i: (0, 0))],
        out_specs=pl.BlockSpec((8, BE), lambda i: (0, i)),
        out_shape=jax.ShapeDtypeStruct((8, rows), F32),
        interpret=False,
    )(xs, xd, ea, w1T, b1, w2T, b2, m1T, bm1, m2T, bm2, m3, bm3)


def _pad2(w, r, c):
    return jnp.zeros((r, c), F32).at[:w.shape[0], :w.shape[1]].set(w)


def _pad_cat3(w, r):
    z = jnp.zeros((r, 3 * HP), F32)
    for t in range(3):
        z = z.at[:w.shape[0], HP * t:HP * t + H].set(w[:, H * t:H * t + H])
    return z


def _padb(b, n=HP):
    return jnp.zeros((1, n), F32).at[0, :b.shape[0]].set(b)


def kernel(x, edge_index, edge_attr, W_node, b_node, W_edge, b_edge,
           W1, b1, W2, b2, We1, be1, We2, be2, gamma, beta,
           Wm1, bm1, Wm2, bm2, Wm3, bm3):
    pad = EP - E
    padi = jnp.arange(pad, dtype=jnp.int32)
    src_flat = jnp.concatenate([edge_index[0], padi % N])
    dst_flat = jnp.concatenate([edge_index[1], N + (padi % (NP - N))])
    dst_gf = jnp.concatenate([edge_index[1], padi % N])
    ea_t = jnp.zeros((DE, EP), F32).at[:, :E].set(edge_attr.T)

    wnT = _pad2(W_node, HP, DF).T
    weT = _pad2(W_edge, HP, DE).T
    bnp = _padb(b_node)
    bep = _padb(b_edge)

    EPH = EP // 2
    x0 = _proj_node(x, wnT, bnp)
    ea0A = _proj_edge(ea_t, weT, bep, hoff=0, rows=EPH)
    ea0B = _proj_edge(ea_t, weT, bep, hoff=EPH, rows=EPH)

    aggA = _sc_msg_scatter(x0, ea0A, src_flat, dst_flat, gather_x=True,
                           idxoff=0, rows=EPH)
    aggB = _sc_msg_scatter(x0, ea0B, src_flat, dst_flat, gather_x=True,
                           idxoff=EPH, rows=EPH, dep=aggA)
    x1 = _node_mlp(x0, aggA, aggB, _pad2(W1[0], HP, HP).T, _padb(b1[0]),
                   _pad2(W2[0], HP, HP).T, _padb(b2[0]),
                   _padb(gamma[0]), _padb(beta[0]))

    xs1A, xd1A = _sc_gather2(x1, src_flat, dst_gf, off=0, rows=EPH)
    xs1B, xd1B = _sc_gather2(x1, src_flat, dst_gf, off=EPH, rows=EPH,
                             dep=xs1A)
    we1T = _pad_cat3(We1[0], HP).T
    we2T = _pad2(We2[0], HP, HP).T
    ea1A = _edge_mlp(xs1A, xd1A, ea0A, we1T, _padb(be1[0]),
                     we2T, _padb(be2[0]))
    ea1B = _edge_mlp(xs1B, xd1B, ea0B, we1T, _padb(be1[0]),
                     we2T, _padb(be2[0]))

    agg2A = _sc_msg_scatter(xs1A, ea1A, src_flat, dst_flat, gather_x=False,
                            idxoff=0, rows=EPH, dep=xs1B)
    agg2B = _sc_msg_scatter(xs1B, ea1B, src_flat, dst_flat, gather_x=False,
                            idxoff=EPH, rows=EPH, dep=agg2A)
    x2 = _node_mlp(x1, agg2A, agg2B, _pad2(W1[1], HP, HP).T, _padb(b1[1]),
                   _pad2(W2[1], HP, HP).T, _padb(b2[1]),
                   _padb(gamma[1]), _padb(beta[1]))

    m1T = _pad_cat3(Wm1, HP).T
    m2T = _pad2(Wm2, HP, HP).T
    m3 = _pad2(Wm3, 8, HP)
    bm3p = jnp.zeros((8, 1), F32).at[:2, 0].set(bm3)
    we1T = _pad_cat3(We1[1], HP).T
    we2T = _pad2(We2[1], HP, HP).T

    xs2A, xd2A = _sc_gather2(x2, src_flat, dst_gf, off=0, rows=EPH)
    xs2B, xd2B = _sc_gather2(x2, src_flat, dst_gf, off=EPH, rows=EPH,
                             dep=xs2A)
    outA = _final_mlp(xs2A, xd2A, ea1A, we1T, _padb(be1[1]),
                      we2T, _padb(be2[1]),
                      m1T, _padb(bm1), m2T, _padb(bm2), m3, bm3p,
                      rows=EPH)
    outB = _final_mlp(xs2B, xd2B, ea1B, we1T, _padb(be1[1]),
                      we2T, _padb(be2[1]),
                      m1T, _padb(bm1), m2T, _padb(bm2), m3, bm3p,
                      rows=EPH)
    oA = outA[:2, :].T
    oB = outB[:2, :E - EPH].T
    return jnp.concatenate([oA, oB], axis=0)
